# Initial kernel scaffold; baseline (speedup 1.0000x reference)
import jax
import jax.numpy as jnp
from jax.experimental import pallas as pl


def kernel(x, edge_index, batch, W1, a_s1, a_d1, b1, W2, a_s2, a_d2, b2, aw, ab, fc1_w, fc1_b, bn_g, bn_b, fc2_w, fc2_b):
    raise NotImplementedError("write your pallas kernel here")


# speedup vs baseline: 23.6890x; 23.6890x over previous
import jax
import jax.numpy as jnp
from jax import lax
from jax.experimental import pallas as pl
from jax.experimental.pallas import tpu as pltpu
from jax.experimental.pallas import tpu_sc as plsc

N = 10000
NP = 10240
D = 128
G = 256
NC = 2
NS = 16
NT = NC * NS
EB = 128
NB = 79
EPT = NB * EB
E_REAL = 320000
RPT = NP // NS
BLK = 1024


def _prologue_body(x_ref, w_ref, as_ref, ad_ref, h_ref, av_ref):
    h = jnp.dot(x_ref[...], w_ref[...], preferred_element_type=jnp.float32)
    h_ref[...] = h
    asv = jnp.sum(h * as_ref[...], axis=1, keepdims=True)
    adv = jnp.sum(h * ad_ref[...], axis=1, keepdims=True)
    av_ref[...] = jnp.concatenate([asv, adv], axis=1)


def _tc_prologue(x, W, a_s, a_d):
    return pl.pallas_call(
        _prologue_body,
        grid=(NP // BLK,),
        in_specs=[
            pl.BlockSpec((BLK, D), lambda i: (i, 0)),
            pl.BlockSpec((D, D), lambda i: (0, 0)),
            pl.BlockSpec((D,), lambda i: (0,)),
            pl.BlockSpec((D,), lambda i: (0,)),
        ],
        out_specs=[
            pl.BlockSpec((BLK, D), lambda i: (i, 0)),
            pl.BlockSpec((BLK, 2), lambda i: (i, 0)),
        ],
        out_shape=[
            jax.ShapeDtypeStruct((NP, D), jnp.float32),
            jax.ShapeDtypeStruct((NP, 2), jnp.float32),
        ],
    )(x, W, a_s, a_d)


def _combine_body(num_ref, den_ref, hp_ref, av_ref, b_ref, w_ref, as_ref,
                  ad_ref, h_ref, av2_ref):
    av = av_ref[...]
    es = av[:, 0:1] + av[:, 1:2]
    ws = jnp.exp(jnp.where(es >= 0.0, es, es * 0.2))
    den = (den_ref[0] + den_ref[1])[:, None] + ws
    hp = hp_ref[...]
    num = num_ref[0] + num_ref[1] + ws * hp
    out = num / den + b_ref[...]
    h1 = jnp.where(out >= 0.0, out, out * 0.01)
    h = jnp.dot(h1, w_ref[...], preferred_element_type=jnp.float32)
    h_ref[...] = h
    asv = jnp.sum(h * as_ref[...], axis=1, keepdims=True)
    adv = jnp.sum(h * ad_ref[...], axis=1, keepdims=True)
    av2_ref[...] = jnp.concatenate([asv, adv], axis=1)


def _tc_combine_prologue(num, den, hp, av, b, W, a_s, a_d):
    return pl.pallas_call(
        _combine_body,
        grid=(NP // BLK,),
        in_specs=[
            pl.BlockSpec((NC, BLK, D), lambda i: (0, i, 0)),
            pl.BlockSpec((NC, BLK), lambda i: (0, i)),
            pl.BlockSpec((BLK, D), lambda i: (i, 0)),
            pl.BlockSpec((BLK, 2), lambda i: (i, 0)),
            pl.BlockSpec((D,), lambda i: (0,)),
            pl.BlockSpec((D, D), lambda i: (0, 0)),
            pl.BlockSpec((D,), lambda i: (0,)),
            pl.BlockSpec((D,), lambda i: (0,)),
        ],
        out_specs=[
            pl.BlockSpec((BLK, D), lambda i: (i, 0)),
            pl.BlockSpec((BLK, 2), lambda i: (i, 0)),
        ],
        out_shape=[
            jax.ShapeDtypeStruct((NP, D), jnp.float32),
            jax.ShapeDtypeStruct((NP, 2), jnp.float32),
        ],
    )(num, den, hp, av, b, W, a_s, a_d)


def _head_body(num_ref, den_ref, hp_ref, av_ref, b_ref, aw_ref, ab_ref,
               batch_ref, fc1w_ref, fc1b_ref, bng_ref, bnb_ref, fc2w_ref,
               fc2b_ref, z_ref):
    av = av_ref[...]
    es = av[:, 0:1] + av[:, 1:2]
    ws = jnp.exp(jnp.where(es >= 0.0, es, es * 0.2))
    den = (den_ref[0] + den_ref[1])[:, None] + ws
    hp = hp_ref[...]
    num = num_ref[0] + num_ref[1] + ws * hp
    out = num / den + b_ref[...]
    h = jnp.where(out >= 0.0, out, out * 0.01)
    s = jnp.dot(h, aw_ref[...], preferred_element_type=jnp.float32) + ab_ref[...]
    valid = lax.broadcasted_iota(jnp.int32, (NP, 1), 0) < N
    m = jnp.max(jnp.where(valid, s, -jnp.inf))
    ex = jnp.where(valid, jnp.exp(s - m), 0.0)
    attn = ex / jnp.sum(ex)
    hw = attn * h
    seg = (lax.broadcasted_iota(jnp.int32, (G, NP), 0) == batch_ref[...]
           ).astype(jnp.float32)
    g = jnp.dot(seg, hw, preferred_element_type=jnp.float32)
    z = jnp.dot(g, fc1w_ref[...], preferred_element_type=jnp.float32) + fc1b_ref[...]
    mean = jnp.mean(z, axis=0, keepdims=True)
    var = jnp.mean((z - mean) ** 2, axis=0, keepdims=True)
    z = bng_ref[...] * (z - mean) / jnp.sqrt(var + 1e-5) + bnb_ref[...]
    z = jnp.where(z >= 0.0, z, z * 0.01)
    z_ref[...] = jnp.dot(z, fc2w_ref[...], preferred_element_type=jnp.float32) + fc2b_ref[...]


def _tc_head(num, den, hp, av, b, aw, ab, batch2d, fc1_w, fc1_b, bn_g, bn_b,
             fc2_w, fc2_b):
    return pl.pallas_call(
        _head_body,
        out_shape=jax.ShapeDtypeStruct((G, D), jnp.float32),
    )(num, den, hp, av, b, aw, ab, batch2d, fc1_w, fc1_b, bn_g, bn_b,
      fc2_w, fc2_b)


def _sc_edge_body(h_hbm, av_hbm, src_hbm, dst_hbm, zrow_hbm,
                  num_hbm, den_hbm,
                  av_v, src_bv, dst_bv, w_v, rows_v, sh_num, sh_den, sem):
    c = lax.axis_index("c")
    s = lax.axis_index("s")
    wid = c * NS + s

    pltpu.sync_copy(av_hbm, av_v)

    pltpu.sync_copy(zrow_hbm.at[pl.ds(0, RPT)],
                    sh_num.at[pl.ds(s * RPT, RPT)])
    for i in range(EB // 16):
        w_v[pl.ds(i * 16, 16)] = jnp.zeros((16,), jnp.float32)
    for k in range(RPT // EB):
        pltpu.sync_copy(w_v, sh_den.at[pl.ds(s * RPT + k * EB, EB)])
    plsc.subcore_barrier()

    def _block(b, carry):
        pltpu.sync_copy(src_hbm.at[wid, b], src_bv.at[0])
        pltpu.sync_copy(dst_hbm.at[wid, b], dst_bv.at[0])
        cp = pltpu.async_copy(h_hbm.at[src_bv.at[0]], rows_v, sem)
        for i in range(EB // 16):
            src16 = src_bv[0, pl.ds(i * 16, 16)]
            dst16 = dst_bv[0, pl.ds(i * 16, 16)]
            sa = plsc.load_gather(av_v, [src16 * 2])
            da = plsc.load_gather(av_v, [dst16 * 2 + 1])
            e = sa + da
            e = jnp.where(e >= 0.0, e, e * 0.2)
            wv = jnp.exp(e)
            gid = wid * EPT + b * EB + i * 16 + lax.iota(jnp.int32, 16)
            wv = jnp.where(gid < E_REAL, wv, 0.0)
            w_v[pl.ds(i * 16, 16)] = wv
        pltpu.sync_copy(w_v, sh_den.at[dst_bv.at[0]], add=True)
        cp.wait()

        def _scale(g, carry2):
            w16 = w_v[pl.ds(g * 16, 16)]
            for j in range(16):
                wj = lax.gather(
                    w16,
                    jnp.full((16, 1), j, jnp.int32),
                    lax.GatherDimensionNumbers(
                        offset_dims=(), collapsed_slice_dims=(0,),
                        start_index_map=(0,)),
                    slice_sizes=(1,),
                    mode=lax.GatherScatterMode.PROMISE_IN_BOUNDS)
                row = g * 16 + j
                for cc in range(8):
                    sl = pl.ds(cc * 16, 16)
                    rows_v[row, sl] = rows_v[row, sl] * wj
            return carry2
        lax.fori_loop(0, EB // 16, _scale, 0)

        pltpu.sync_copy(rows_v, sh_num.at[dst_bv.at[0]], add=True)
        return carry
    lax.fori_loop(0, NB, _block, 0)

    plsc.subcore_barrier()
    pltpu.sync_copy(sh_den.at[pl.ds(s * RPT, RPT)],
                    den_hbm.at[c, pl.ds(s * RPT, RPT)])
    pltpu.sync_copy(sh_num.at[pl.ds(s * RPT, RPT)],
                    num_hbm.at[c, pl.ds(s * RPT, RPT)])


_sc_edge = pl.kernel(
    _sc_edge_body,
    out_type=[
        jax.ShapeDtypeStruct((NC, NP, D), jnp.float32),
        jax.ShapeDtypeStruct((NC, NP), jnp.float32),
    ],
    mesh=plsc.VectorSubcoreMesh(core_axis_name="c", subcore_axis_name="s",
                                num_cores=NC, num_subcores=NS),
    compiler_params=pltpu.CompilerParams(needs_layout_passes=False),
    scratch_types=[
        pltpu.VMEM((2 * NP,), jnp.float32),
        pltpu.VMEM((1, EB), jnp.int32),
        pltpu.VMEM((1, EB), jnp.int32),
        pltpu.VMEM((EB,), jnp.float32),
        pltpu.VMEM((EB, D), jnp.float32),
        pltpu.VMEM_SHARED((NP, D), jnp.float32),
        pltpu.VMEM_SHARED((NP,), jnp.float32),
        pltpu.SemaphoreType.DMA,
    ],
)


def kernel(x, edge_index, batch, W1, a_s1, a_d1, b1, W2, a_s2, a_d2, b2, aw,
           ab, fc1_w, fc1_b, bn_g, bn_b, fc2_w, fc2_b):
    src = edge_index[0].astype(jnp.int32)
    dst = edge_index[1].astype(jnp.int32)
    pad = NT * EPT - E_REAL
    src_p = jnp.pad(src, (0, pad)).reshape(NT, NB, EB)
    dst_p = jnp.pad(dst, (0, pad)).reshape(NT, NB, EB)
    zrow = jnp.zeros((NP, D), jnp.float32)
    xp = jnp.pad(x, ((0, NP - N), (0, 0)))
    batch2d = jnp.pad(batch.astype(jnp.int32), (0, NP - N),
                      constant_values=G).reshape(1, NP)

    h1p, av1 = _tc_prologue(xp, W1, a_s1, a_d1)
    num1, den1 = _sc_edge(h1p, av1.reshape(-1), src_p, dst_p, zrow)
    h2p, av2 = _tc_combine_prologue(num1, den1, h1p, av1, b1, W2, a_s2, a_d2)
    num2, den2 = _sc_edge(h2p, av2.reshape(-1), src_p, dst_p, zrow)
    return _tc_head(num2, den2, h2p, av2, b2, aw, ab, batch2d, fc1_w, fc1_b,
                    bn_g, bn_b, fc2_w, fc2_b)
